# baseline (speedup 1.0000x reference)
import functools
import jax
import jax.numpy as jnp
from jax import lax
from jax.experimental import pallas as pl
from jax.experimental.pallas import tpu as pltpu
from jax.experimental.pallas import tpu_sc as plsc

N = 10000
E = 160000
H = 64
LH = 64

NP = 10240
NC = 2
NS = 16
NW = NC * NS
CH = 128
EPW = 5120
E2 = NW * EPW
T = 1000
HP = 128
BN_S = 1.0 / (1.0 + 1e-5) ** 0.5

@functools.lru_cache(maxsize=None)
def _mesh():
    return plsc.VectorSubcoreMesh(
        core_axis_name="c", subcore_axis_name="s",
        num_cores=NC, num_subcores=NS)


def _wid():
    return lax.axis_index("s") * NC + lax.axis_index("c")


def _zero_1d(ref, n):
    z = jnp.zeros((16,), ref.dtype)

    def body(i, carry):
        ref[pl.ds(i * 16, 16)] = z
        return carry

    lax.fori_loop(0, n // 16, body, 0)


def _zero_2d(ref, rows, width):
    z = jnp.zeros((16,), ref.dtype)

    def body(i, carry):
        for j in range(width // 16):
            ref[i, pl.ds(16 * j, 16)] = z
        return carry

    lax.fori_loop(0, rows, body, 0)


@functools.lru_cache(maxsize=None)
def _build_sc_deg():
    return pl.kernel(
        _sc_deg_body,
        out_type=jax.ShapeDtypeStruct((NW, NP), jnp.float32),
        mesh=_mesh(),
        compiler_params=pltpu.CompilerParams(needs_layout_passes=False),
        scratch_types=[
            pltpu.VMEM((EPW,), jnp.int32),
            pltpu.VMEM((NP,), jnp.float32),
        ],
    )


def _sc_deg(dst2):
    return _build_sc_deg()(dst2)


def _sc_deg_body(dst_hbm, out_hbm, idx_v, acc_v):
    wid = _wid()
    pltpu.sync_copy(dst_hbm.at[pl.ds(wid * EPW, EPW)], idx_v)
    _zero_1d(acc_v, NP)
    ones = jnp.ones((16,), jnp.float32)

    def step(i, carry):
        idx = idx_v[pl.ds(i * 16, 16)]
        plsc.addupdate_scatter(acc_v, [idx], ones)
        return carry

    lax.fori_loop(0, EPW // 16, step, 0)
    pltpu.sync_copy(acc_v, out_hbm.at[wid])


@functools.lru_cache(maxsize=None)
def _build_sc_agg1():
    return pl.kernel(
        _sc_agg1_body,
        out_type=jax.ShapeDtypeStruct((NW, NP), jnp.float32),
        mesh=_mesh(),
        compiler_params=pltpu.CompilerParams(needs_layout_passes=False),
        scratch_types=[
            pltpu.VMEM((EPW,), jnp.int32),
            pltpu.VMEM((EPW,), jnp.int32),
            pltpu.VMEM((NP,), jnp.float32),
            pltpu.VMEM((NP,), jnp.float32),
        ],
    )


def _sc_agg1(src2, dst2, u1):
    return _build_sc_agg1()(src2, dst2, u1)


def _sc_agg1_body(src_hbm, dst_hbm, u1_hbm, out_hbm, src_v, dst_v, u1_v, acc_v):
    wid = _wid()
    pltpu.sync_copy(src_hbm.at[pl.ds(wid * EPW, EPW)], src_v)
    pltpu.sync_copy(dst_hbm.at[pl.ds(wid * EPW, EPW)], dst_v)
    pltpu.sync_copy(u1_hbm, u1_v)
    _zero_1d(acc_v, NP)

    def step(i, carry):
        s = src_v[pl.ds(i * 16, 16)]
        d = dst_v[pl.ds(i * 16, 16)]
        vals = plsc.load_gather(u1_v, [s])
        plsc.addupdate_scatter(acc_v, [d], vals)
        return carry

    lax.fori_loop(0, EPW // 16, step, 0)
    pltpu.sync_copy(acc_v, out_hbm.at[wid])


@functools.lru_cache(maxsize=None)
def _build_sc_agg2():
    return pl.kernel(
        _sc_agg2_body,
        out_type=jax.ShapeDtypeStruct((NC, NP, HP), jnp.float32),
        mesh=_mesh(),
        compiler_params=pltpu.CompilerParams(needs_layout_passes=False),
        scratch_types=[
            pltpu.VMEM((CH,), jnp.int32),
            pltpu.VMEM((CH,), jnp.int32),
            pltpu.VMEM((CH, HP), jnp.float32),
            pltpu.VMEM((CH, HP), jnp.float32),
            pltpu.VMEM_SHARED((NP, HP), jnp.float32),
            pltpu.SemaphoreType.DMA,
        ],
    )


def _sc_agg2(src2, dst2, u2):
    return _build_sc_agg2()(src2, dst2, u2)


def _sc_agg2_body(src_hbm, dst_hbm, u2_hbm, out_hbm, sidx_v, didx_v, rows_v,
                  zbuf_v, acc_sp, sem):
    cid = lax.axis_index("c")
    sid = lax.axis_index("s")
    wid = sid * NC + cid
    rows_per_tile = NP // NS
    _zero_2d(zbuf_v, CH, HP)
    for j in range(rows_per_tile // CH):
        pltpu.sync_copy(zbuf_v, acc_sp.at[pl.ds(sid * rows_per_tile + j * CH, CH)])
    plsc.subcore_barrier()

    def step(i, carry):
        base = wid * EPW + i * CH
        pltpu.sync_copy(src_hbm.at[pl.ds(base, CH)], sidx_v)
        pltpu.sync_copy(dst_hbm.at[pl.ds(base, CH)], didx_v)
        pltpu.async_copy(u2_hbm.at[sidx_v], rows_v, sem).wait()
        pltpu.sync_copy(rows_v, acc_sp.at[didx_v], add=True)
        return carry

    lax.fori_loop(0, EPW // CH, step, 0)
    plsc.subcore_barrier()
    pltpu.sync_copy(
        acc_sp.at[pl.ds(sid * rows_per_tile, rows_per_tile)],
        out_hbm.at[cid, pl.ds(sid * rows_per_tile, rows_per_tile)],
    )


@functools.lru_cache(maxsize=None)
def _build_sc_gather():
    return pl.kernel(
        _sc_gather_body,
        out_type=(
            jax.ShapeDtypeStruct((E2, HP), jnp.float32),
            jax.ShapeDtypeStruct((E2, HP), jnp.float32),
        ),
        mesh=_mesh(),
        compiler_params=pltpu.CompilerParams(needs_layout_passes=False),
        scratch_types=[
            pltpu.VMEM((CH,), jnp.int32),
            pltpu.VMEM((CH,), jnp.int32),
            pltpu.VMEM((CH, HP), jnp.float32),
            pltpu.VMEM((CH, HP), jnp.float32),
            pltpu.SemaphoreType.DMA,
            pltpu.SemaphoreType.DMA,
        ],
    )


def _sc_gather(src2, dst2, h2):
    return _build_sc_gather()(src2, dst2, h2)


def _sc_gather_body(src_hbm, dst_hbm, h2_hbm, snd_hbm, rcv_hbm, sidx_v, didx_v,
                    srows_v, drows_v, sem1, sem2):
    wid = _wid()

    def step(i, carry):
        base = wid * EPW + i * CH
        pltpu.sync_copy(src_hbm.at[pl.ds(base, CH)], sidx_v)
        pltpu.sync_copy(dst_hbm.at[pl.ds(base, CH)], didx_v)
        c1 = pltpu.async_copy(h2_hbm.at[sidx_v], srows_v, sem1)
        c2 = pltpu.async_copy(h2_hbm.at[didx_v], drows_v, sem2)
        c1.wait()
        c2.wait()
        pltpu.sync_copy(srows_v, snd_hbm.at[pl.ds(base, CH)])
        pltpu.sync_copy(drows_v, rcv_hbm.at[pl.ds(base, CH)])
        return carry

    lax.fori_loop(0, EPW // CH, step, 0)


def _tc_a_body(degp_ref, x_ref, dinv_ref, u1_ref):
    deg = jnp.sum(degp_ref[...], axis=0, keepdims=True) + 1.0
    dinv = lax.rsqrt(deg)
    dinv_ref[...] = dinv
    u1_ref[...] = x_ref[...] * dinv


def _tc_b1_body(aggp_ref, u1_ref, dinv_ref, y_ref):
    agg = jnp.sum(aggp_ref[...], axis=0, keepdims=True)
    y_ref[...] = dinv_ref[...] * (agg + u1_ref[...])


def _tc_b2_body(y_ref, dinv_ref, w1_ref, b1_ref, g1_ref, be1_ref, w2_ref,
                u2_ref):
    g1s = g1_ref[...] * BN_S
    h1 = jnp.maximum((y_ref[...] * w1_ref[...] + b1_ref[...]) * g1s
                     + be1_ref[...], 0.0)
    u2 = dinv_ref[...] * jnp.dot(
        h1, w2_ref[...], preferred_element_type=jnp.float32)
    u2_ref[...] = jnp.concatenate(
        [u2, jnp.zeros((NP, HP - H), jnp.float32)], axis=1)


def _tc_c_body(p0_ref, p1_ref, u2_ref, dinv_ref, b2_ref, g2_ref, be2_ref,
               h2_ref):
    g2s = g2_ref[...] * BN_S
    agg = (p0_ref[...] + p1_ref[...] + u2_ref[...])[:, :H]
    h2 = jnp.maximum(
        (dinv_ref[...] * agg + b2_ref[...]) * g2s + be2_ref[...], 0.0)
    h2_ref[...] = jnp.concatenate(
        [h2, jnp.zeros((NP, HP - H), jnp.float32)], axis=1)


def _lstm_body(snd_ref, rcv_ref, ea_ref,
               ws_ref, wr_ref, we_ref, b_ref, whh_ref,
               wl1_ref, bl1_ref, wl2_ref, bl2_ref,
               out_ref,
               g_s, hs_s, h_s, c_s):
    g_s[...] = (
        jnp.dot(snd_ref[...], ws_ref[...], preferred_element_type=jnp.float32)
        + jnp.dot(rcv_ref[...], wr_ref[...],
                  preferred_element_type=jnp.float32)
        + jnp.dot(ea_ref[...], we_ref[...],
                  preferred_element_type=jnp.float32)
        + b_ref[...])

    @pl.when(pl.program_id(0) == 0)
    def _():
        h_s[...] = jnp.zeros((1, LH), jnp.float32)
        c_s[...] = jnp.zeros((1, LH), jnp.float32)

    whh = whh_ref[...]

    def step(t, carry):
        h, c = carry
        z = g_s[pl.ds(t, 1), :] + jnp.dot(
            h, whh, preferred_element_type=jnp.float32)
        zi = lax.slice(z, (0, 0), (1, LH))
        zf = lax.slice(z, (0, LH), (1, 2 * LH))
        zg = lax.slice(z, (0, 2 * LH), (1, 3 * LH))
        zo = lax.slice(z, (0, 3 * LH), (1, 4 * LH))
        ig = 0.5 * jnp.tanh(0.5 * zi) + 0.5
        fg = 0.5 * jnp.tanh(0.5 * zf) + 0.5
        gg = jnp.tanh(zg)
        og = 0.5 * jnp.tanh(0.5 * zo) + 0.5
        c = fg * c + ig * gg
        h = og * jnp.tanh(c)
        hs_s[pl.ds(t, 1), :] = h
        return (h, c)

    h, c = lax.fori_loop(0, T, step, (h_s[...], c_s[...]))
    h_s[...] = h
    c_s[...] = c

    m = jnp.maximum(
        jnp.dot(hs_s[...], wl1_ref[...], preferred_element_type=jnp.float32)
        + bl1_ref[...], 0.0)
    out_ref[...] = (jnp.dot(m, wl2_ref[...], preferred_element_type=jnp.float32)
                    + bl2_ref[...])


def _full_spec(shape):
    return pl.BlockSpec(shape, lambda i: tuple(0 for _ in shape))


def kernel(x, edge_index, edge_attr, W1, b1, g1, be1, W2, b2, g2, be2, Wih,
           Whh, bih, bhh, Wl1, bl1, Wl2, bl2):
    f32 = jnp.float32
    src = edge_index[0].astype(jnp.int32)
    dst = edge_index[1].astype(jnp.int32)
    padi = jnp.full((E2 - E,), N, jnp.int32)
    src2 = jnp.concatenate([src, padi])
    dst2 = jnp.concatenate([dst, padi])

    x_row = jnp.zeros((1, NP), f32).at[0, :N].set(x[:, 0])

    degp = _sc_deg(dst2)
    dinv_row, u1_row = pl.pallas_call(
        _tc_a_body,
        out_shape=(jax.ShapeDtypeStruct((1, NP), f32),
                   jax.ShapeDtypeStruct((1, NP), f32)),
    )(degp, x_row)

    aggp1 = _sc_agg1(src2, dst2, u1_row.reshape(NP))
    y_row = pl.pallas_call(
        _tc_b1_body,
        out_shape=jax.ShapeDtypeStruct((1, NP), f32),
    )(aggp1, u1_row, dinv_row)

    y_col = y_row.reshape(NP, 1)
    dinv_col = dinv_row.reshape(NP, 1)

    u2 = pl.pallas_call(
        _tc_b2_body,
        out_shape=jax.ShapeDtypeStruct((NP, HP), f32),
    )(y_col, dinv_col, W1.reshape(1, H), b1.reshape(1, H), g1.reshape(1, H),
      be1.reshape(1, H), W2)

    aggp2 = _sc_agg2(src2, dst2, u2)
    h2 = pl.pallas_call(
        _tc_c_body,
        out_shape=jax.ShapeDtypeStruct((NP, HP), f32),
    )(aggp2[0], aggp2[1], u2, dinv_col, b2.reshape(1, H), g2.reshape(1, H),
      be2.reshape(1, H))

    snd2, rcv2 = _sc_gather(src2, dst2, h2)
    snd = lax.slice(snd2, (0, 0), (E, HP))
    rcv = lax.slice(rcv2, (0, 0), (E, HP))

    WihT = Wih.T
    ws = jnp.concatenate(
        [lax.slice(WihT, (0, 0), (H, 4 * LH)),
         jnp.zeros((HP - H, 4 * LH), jnp.float32)])
    wr = jnp.concatenate(
        [lax.slice(WihT, (H, 0), (2 * H, 4 * LH)),
         jnp.zeros((HP - H, 4 * LH), jnp.float32)])
    we = lax.slice(WihT, (2 * H, 0), (2 * H + 2, 4 * LH))
    b = (bih + bhh).reshape(1, 4 * LH)
    whh = Whh.T

    grid = E // T
    edge_spec = pl.BlockSpec((T, HP), lambda i: (i, 0))
    out = pl.pallas_call(
        _lstm_body,
        grid=(grid,),
        in_specs=[
            edge_spec, edge_spec, pl.BlockSpec((T, 2), lambda i: (i, 0)),
            _full_spec((HP, 4 * LH)), _full_spec((HP, 4 * LH)),
            _full_spec((2, 4 * LH)), _full_spec((1, 4 * LH)),
            _full_spec((LH, 4 * LH)),
            _full_spec((LH, LH // 2)), _full_spec((1, LH // 2)),
            _full_spec((LH // 2, 1)), _full_spec((1, 1)),
        ],
        out_specs=pl.BlockSpec((T, 1), lambda i: (i, 0)),
        out_shape=jax.ShapeDtypeStruct((E, 1), f32),
        scratch_shapes=[
            pltpu.VMEM((T, 4 * LH), f32), pltpu.VMEM((T, LH), f32),
            pltpu.VMEM((1, LH), f32), pltpu.VMEM((1, LH), f32),
        ],
        compiler_params=pltpu.CompilerParams(
            dimension_semantics=("arbitrary",)),
    )(snd, rcv, edge_attr, ws, wr, we, b, whh,
      Wl1.T, bl1.reshape(1, LH // 2), Wl2.T, bl2.reshape(1, 1))

    return out.reshape(-1)


# speedup vs baseline: 2.3976x; 2.3976x over previous
import functools
import jax
import jax.numpy as jnp
from jax import lax
from jax.experimental import pallas as pl
from jax.experimental.pallas import tpu as pltpu
from jax.experimental.pallas import tpu_sc as plsc

N = 10000
E = 160000
H = 64
LH = 64

NP = 10240
NC = 2
NS = 16
NW = NC * NS
CH = 128
EPW = 5120
E2 = NW * EPW
T = 1000
HP = 128
BN_S = 1.0 / (1.0 + 1e-5) ** 0.5

@functools.lru_cache(maxsize=None)
def _mesh():
    return plsc.VectorSubcoreMesh(
        core_axis_name="c", subcore_axis_name="s",
        num_cores=NC, num_subcores=NS)


def _wid():
    return lax.axis_index("s") * NC + lax.axis_index("c")


def _zero_1d(ref, n):
    z = jnp.zeros((16,), ref.dtype)

    def body(i, carry):
        ref[pl.ds(i * 16, 16)] = z
        return carry

    lax.fori_loop(0, n // 16, body, 0)


def _zero_2d(ref, rows, width):
    z = jnp.zeros((16,), ref.dtype)

    def body(i, carry):
        for j in range(width // 16):
            ref[i, pl.ds(16 * j, 16)] = z
        return carry

    lax.fori_loop(0, rows, body, 0)


@functools.lru_cache(maxsize=None)
def _build_sc_deg():
    return pl.kernel(
        _sc_deg_body,
        out_type=jax.ShapeDtypeStruct((NW, NP), jnp.float32),
        mesh=_mesh(),
        compiler_params=pltpu.CompilerParams(needs_layout_passes=False),
        scratch_types=[
            pltpu.VMEM((EPW,), jnp.int32),
            pltpu.VMEM((NP,), jnp.float32),
        ],
    )


def _sc_deg(dst2):
    return _build_sc_deg()(dst2)


def _sc_deg_body(dst_hbm, out_hbm, idx_v, acc_v):
    wid = _wid()
    pltpu.sync_copy(dst_hbm.at[pl.ds(wid * EPW, EPW)], idx_v)
    _zero_1d(acc_v, NP)
    ones = jnp.ones((16,), jnp.float32)

    def step(i, carry):
        idx = idx_v[pl.ds(i * 16, 16)]
        plsc.addupdate_scatter(acc_v, [idx], ones)
        return carry

    lax.fori_loop(0, EPW // 16, step, 0)
    pltpu.sync_copy(acc_v, out_hbm.at[wid])


@functools.lru_cache(maxsize=None)
def _build_sc_agg1():
    return pl.kernel(
        _sc_agg1_body,
        out_type=jax.ShapeDtypeStruct((NW, NP), jnp.float32),
        mesh=_mesh(),
        compiler_params=pltpu.CompilerParams(needs_layout_passes=False),
        scratch_types=[
            pltpu.VMEM((EPW,), jnp.int32),
            pltpu.VMEM((EPW,), jnp.int32),
            pltpu.VMEM((NP,), jnp.float32),
            pltpu.VMEM((NP,), jnp.float32),
        ],
    )


def _sc_agg1(src2, dst2, u1):
    return _build_sc_agg1()(src2, dst2, u1)


def _sc_agg1_body(src_hbm, dst_hbm, u1_hbm, out_hbm, src_v, dst_v, u1_v, acc_v):
    wid = _wid()
    pltpu.sync_copy(src_hbm.at[pl.ds(wid * EPW, EPW)], src_v)
    pltpu.sync_copy(dst_hbm.at[pl.ds(wid * EPW, EPW)], dst_v)
    pltpu.sync_copy(u1_hbm, u1_v)
    _zero_1d(acc_v, NP)

    def step(i, carry):
        s = src_v[pl.ds(i * 16, 16)]
        d = dst_v[pl.ds(i * 16, 16)]
        vals = plsc.load_gather(u1_v, [s])
        plsc.addupdate_scatter(acc_v, [d], vals)
        return carry

    lax.fori_loop(0, EPW // 16, step, 0)
    pltpu.sync_copy(acc_v, out_hbm.at[wid])


@functools.lru_cache(maxsize=None)
def _build_sc_agg2():
    return pl.kernel(
        _sc_agg2_body,
        out_type=jax.ShapeDtypeStruct((NC, NP, HP), jnp.float32),
        mesh=_mesh(),
        compiler_params=pltpu.CompilerParams(needs_layout_passes=False),
        scratch_types=[
            pltpu.VMEM((CH,), jnp.int32),
            pltpu.VMEM((CH,), jnp.int32),
            pltpu.VMEM((CH, HP), jnp.float32),
            pltpu.VMEM((CH, HP), jnp.float32),
            pltpu.VMEM_SHARED((NP, HP), jnp.float32),
            pltpu.SemaphoreType.DMA,
        ],
    )


def _sc_agg2(src2, dst2, u2):
    return _build_sc_agg2()(src2, dst2, u2)


def _sc_agg2_body(src_hbm, dst_hbm, u2_hbm, out_hbm, sidx_v, didx_v, rows_v,
                  zbuf_v, acc_sp, sem):
    cid = lax.axis_index("c")
    sid = lax.axis_index("s")
    wid = sid * NC + cid
    rows_per_tile = NP // NS
    _zero_2d(zbuf_v, CH, HP)
    for j in range(rows_per_tile // CH):
        pltpu.sync_copy(zbuf_v, acc_sp.at[pl.ds(sid * rows_per_tile + j * CH, CH)])
    plsc.subcore_barrier()

    def step(i, carry):
        base = wid * EPW + i * CH
        pltpu.sync_copy(src_hbm.at[pl.ds(base, CH)], sidx_v)
        pltpu.sync_copy(dst_hbm.at[pl.ds(base, CH)], didx_v)
        pltpu.async_copy(u2_hbm.at[sidx_v], rows_v, sem).wait()
        pltpu.sync_copy(rows_v, acc_sp.at[didx_v], add=True)
        return carry

    lax.fori_loop(0, EPW // CH, step, 0)
    plsc.subcore_barrier()
    pltpu.sync_copy(
        acc_sp.at[pl.ds(sid * rows_per_tile, rows_per_tile)],
        out_hbm.at[cid, pl.ds(sid * rows_per_tile, rows_per_tile)],
    )


@functools.lru_cache(maxsize=None)
def _build_sc_gather():
    return pl.kernel(
        _sc_gather_body,
        out_type=(
            jax.ShapeDtypeStruct((E2, HP), jnp.float32),
            jax.ShapeDtypeStruct((E2, HP), jnp.float32),
        ),
        mesh=_mesh(),
        compiler_params=pltpu.CompilerParams(needs_layout_passes=False),
        scratch_types=[
            pltpu.VMEM((CH,), jnp.int32),
            pltpu.VMEM((CH,), jnp.int32),
            pltpu.VMEM((CH, HP), jnp.float32),
            pltpu.VMEM((CH, HP), jnp.float32),
            pltpu.SemaphoreType.DMA,
            pltpu.SemaphoreType.DMA,
        ],
    )


def _sc_gather(src2, dst2, h2):
    return _build_sc_gather()(src2, dst2, h2)


def _sc_gather_body(src_hbm, dst_hbm, h2_hbm, snd_hbm, rcv_hbm, sidx_v, didx_v,
                    srows_v, drows_v, sem1, sem2):
    wid = _wid()

    def step(i, carry):
        base = wid * EPW + i * CH
        pltpu.sync_copy(src_hbm.at[pl.ds(base, CH)], sidx_v)
        pltpu.sync_copy(dst_hbm.at[pl.ds(base, CH)], didx_v)
        c1 = pltpu.async_copy(h2_hbm.at[sidx_v], srows_v, sem1)
        c2 = pltpu.async_copy(h2_hbm.at[didx_v], drows_v, sem2)
        c1.wait()
        c2.wait()
        pltpu.sync_copy(srows_v, snd_hbm.at[pl.ds(base, CH)])
        pltpu.sync_copy(drows_v, rcv_hbm.at[pl.ds(base, CH)])
        return carry

    lax.fori_loop(0, EPW // CH, step, 0)


def _tc_a_body(degp_ref, x_ref, dinv_ref, u1_ref):
    deg = jnp.sum(degp_ref[...], axis=0, keepdims=True) + 1.0
    dinv = lax.rsqrt(deg)
    dinv_ref[...] = dinv
    u1_ref[...] = x_ref[...] * dinv


def _tc_b1_body(aggp_ref, u1_ref, dinv_ref, y_ref):
    agg = jnp.sum(aggp_ref[...], axis=0, keepdims=True)
    y_ref[...] = dinv_ref[...] * (agg + u1_ref[...])


def _tc_b2_body(y_ref, dinv_ref, w1_ref, b1_ref, g1_ref, be1_ref, w2_ref,
                u2_ref):
    g1s = g1_ref[...] * BN_S
    h1 = jnp.maximum((y_ref[...] * w1_ref[...] + b1_ref[...]) * g1s
                     + be1_ref[...], 0.0)
    u2 = dinv_ref[...] * jnp.dot(
        h1, w2_ref[...], preferred_element_type=jnp.float32)
    u2_ref[...] = jnp.concatenate(
        [u2, jnp.zeros((NP, HP - H), jnp.float32)], axis=1)


def _tc_c_body(p0_ref, p1_ref, u2_ref, dinv_ref, b2_ref, g2_ref, be2_ref,
               h2_ref):
    g2s = g2_ref[...] * BN_S
    agg = (p0_ref[...] + p1_ref[...] + u2_ref[...])[:, :H]
    h2 = jnp.maximum(
        (dinv_ref[...] * agg + b2_ref[...]) * g2s + be2_ref[...], 0.0)
    h2_ref[...] = jnp.concatenate(
        [h2, jnp.zeros((NP, HP - H), jnp.float32)], axis=1)


def _lstm_body(snd_ref, rcv_ref, ea_ref,
               ws_ref, wr_ref, we_ref, b_ref,
               wi_ref, wf_ref, wg_ref, wo_ref,
               wl1_ref, bl1_ref, wl2_ref, bl2_ref,
               out_ref,
               gi_s, gf_s, gg_s, go_s, hs_s, h_s, c_s):
    g = (jnp.dot(snd_ref[...], ws_ref[...], preferred_element_type=jnp.float32)
         + jnp.dot(rcv_ref[...], wr_ref[...],
                   preferred_element_type=jnp.float32)
         + jnp.dot(ea_ref[...], we_ref[...],
                   preferred_element_type=jnp.float32)
         + b_ref[...])
    gi_s[...] = lax.slice(g, (0, 0), (T, LH))
    gf_s[...] = lax.slice(g, (0, LH), (T, 2 * LH))
    gg_s[...] = lax.slice(g, (0, 2 * LH), (T, 3 * LH))
    go_s[...] = lax.slice(g, (0, 3 * LH), (T, 4 * LH))

    @pl.when(pl.program_id(0) == 0)
    def _():
        h_s[...] = jnp.zeros((1, LH), jnp.float32)
        c_s[...] = jnp.zeros((1, LH), jnp.float32)

    wi = wi_ref[...]
    wf = wf_ref[...]
    wg = wg_ref[...]
    wo = wo_ref[...]

    def step(t, carry):
        h, c, hb = carry
        zi = gi_s[pl.ds(t, 1), :] + jnp.sum(hb * wi, axis=0, keepdims=True)
        zf = gf_s[pl.ds(t, 1), :] + jnp.sum(hb * wf, axis=0, keepdims=True)
        zg = gg_s[pl.ds(t, 1), :] + jnp.sum(hb * wg, axis=0, keepdims=True)
        zo = go_s[pl.ds(t, 1), :] + jnp.sum(hb * wo, axis=0, keepdims=True)
        ig = 0.5 * jnp.tanh(0.5 * zi) + 0.5
        fg = 0.5 * jnp.tanh(0.5 * zf) + 0.5
        gg = jnp.tanh(zg)
        og = 0.5 * jnp.tanh(0.5 * zo) + 0.5
        c = fg * c + ig * gg
        h = og * jnp.tanh(c)
        hs_s[pl.ds(t, 1), :] = h
        hb = jnp.broadcast_to(jnp.reshape(h, (LH, 1)), (LH, LH))
        return (h, c, hb)

    h, c, _ = lax.fori_loop(
        0, T, step,
        (h_s[...], c_s[...],
         jnp.broadcast_to(jnp.reshape(h_s[...], (LH, 1)), (LH, LH))))
    h_s[...] = h
    c_s[...] = c

    m = jnp.maximum(
        jnp.dot(hs_s[...], wl1_ref[...], preferred_element_type=jnp.float32)
        + bl1_ref[...], 0.0)
    out_ref[...] = (jnp.dot(m, wl2_ref[...], preferred_element_type=jnp.float32)
                    + bl2_ref[...])


def _full_spec(shape):
    return pl.BlockSpec(shape, lambda i: tuple(0 for _ in shape))


def kernel(x, edge_index, edge_attr, W1, b1, g1, be1, W2, b2, g2, be2, Wih,
           Whh, bih, bhh, Wl1, bl1, Wl2, bl2):
    f32 = jnp.float32
    src = edge_index[0].astype(jnp.int32)
    dst = edge_index[1].astype(jnp.int32)
    padi = jnp.full((E2 - E,), N, jnp.int32)
    src2 = jnp.concatenate([src, padi])
    dst2 = jnp.concatenate([dst, padi])

    x_row = jnp.zeros((1, NP), f32).at[0, :N].set(x[:, 0])

    degp = _sc_deg(dst2)
    dinv_row, u1_row = pl.pallas_call(
        _tc_a_body,
        out_shape=(jax.ShapeDtypeStruct((1, NP), f32),
                   jax.ShapeDtypeStruct((1, NP), f32)),
    )(degp, x_row)

    aggp1 = _sc_agg1(src2, dst2, u1_row.reshape(NP))
    y_row = pl.pallas_call(
        _tc_b1_body,
        out_shape=jax.ShapeDtypeStruct((1, NP), f32),
    )(aggp1, u1_row, dinv_row)

    y_col = y_row.reshape(NP, 1)
    dinv_col = dinv_row.reshape(NP, 1)

    u2 = pl.pallas_call(
        _tc_b2_body,
        out_shape=jax.ShapeDtypeStruct((NP, HP), f32),
    )(y_col, dinv_col, W1.reshape(1, H), b1.reshape(1, H), g1.reshape(1, H),
      be1.reshape(1, H), W2)

    aggp2 = _sc_agg2(src2, dst2, u2)
    h2 = pl.pallas_call(
        _tc_c_body,
        out_shape=jax.ShapeDtypeStruct((NP, HP), f32),
    )(aggp2[0], aggp2[1], u2, dinv_col, b2.reshape(1, H), g2.reshape(1, H),
      be2.reshape(1, H))

    snd2, rcv2 = _sc_gather(src2, dst2, h2)
    snd = lax.slice(snd2, (0, 0), (E, HP))
    rcv = lax.slice(rcv2, (0, 0), (E, HP))

    WihT = Wih.T
    ws = jnp.concatenate(
        [lax.slice(WihT, (0, 0), (H, 4 * LH)),
         jnp.zeros((HP - H, 4 * LH), jnp.float32)])
    wr = jnp.concatenate(
        [lax.slice(WihT, (H, 0), (2 * H, 4 * LH)),
         jnp.zeros((HP - H, 4 * LH), jnp.float32)])
    we = lax.slice(WihT, (2 * H, 0), (2 * H + 2, 4 * LH))
    b = (bih + bhh).reshape(1, 4 * LH)
    WhhT = Whh.T
    wi = lax.slice(WhhT, (0, 0), (LH, LH))
    wf = lax.slice(WhhT, (0, LH), (LH, 2 * LH))
    wg = lax.slice(WhhT, (0, 2 * LH), (LH, 3 * LH))
    wo = lax.slice(WhhT, (0, 3 * LH), (LH, 4 * LH))

    grid = E // T
    edge_spec = pl.BlockSpec((T, HP), lambda i: (i, 0))
    out = pl.pallas_call(
        _lstm_body,
        grid=(grid,),
        in_specs=[
            edge_spec, edge_spec, pl.BlockSpec((T, 2), lambda i: (i, 0)),
            _full_spec((HP, 4 * LH)), _full_spec((HP, 4 * LH)),
            _full_spec((2, 4 * LH)), _full_spec((1, 4 * LH)),
            _full_spec((LH, LH)), _full_spec((LH, LH)),
            _full_spec((LH, LH)), _full_spec((LH, LH)),
            _full_spec((LH, LH // 2)), _full_spec((1, LH // 2)),
            _full_spec((LH // 2, 1)), _full_spec((1, 1)),
        ],
        out_specs=pl.BlockSpec((T, 1), lambda i: (i, 0)),
        out_shape=jax.ShapeDtypeStruct((E, 1), f32),
        scratch_shapes=[
            pltpu.VMEM((T, LH), f32), pltpu.VMEM((T, LH), f32),
            pltpu.VMEM((T, LH), f32), pltpu.VMEM((T, LH), f32),
            pltpu.VMEM((T, LH), f32),
            pltpu.VMEM((1, LH), f32), pltpu.VMEM((1, LH), f32),
        ],
        compiler_params=pltpu.CompilerParams(
            dimension_semantics=("arbitrary",)),
    )(snd, rcv, edge_attr, ws, wr, we, b, wi, wf, wg, wo,
      Wl1.T, bl1.reshape(1, LH // 2), Wl2.T, bl2.reshape(1, 1))

    return out.reshape(-1)


# speedup vs baseline: 2.4842x; 1.0361x over previous
import functools
import jax
import jax.numpy as jnp
from jax import lax
from jax.experimental import pallas as pl
from jax.experimental.pallas import tpu as pltpu
from jax.experimental.pallas import tpu_sc as plsc

N = 10000
E = 160000
H = 64
LH = 64

NP = 10240
NC = 2
NS = 16
NW = NC * NS
CH = 128
EPW = 5120
E2 = NW * EPW
T = 1000
HP = 128
BN_S = 1.0 / (1.0 + 1e-5) ** 0.5

@functools.lru_cache(maxsize=None)
def _mesh():
    return plsc.VectorSubcoreMesh(
        core_axis_name="c", subcore_axis_name="s",
        num_cores=NC, num_subcores=NS)


def _wid():
    return lax.axis_index("s") * NC + lax.axis_index("c")


def _zero_1d(ref, n):
    z = jnp.zeros((16,), ref.dtype)

    def body(i, carry):
        ref[pl.ds(i * 16, 16)] = z
        return carry

    lax.fori_loop(0, n // 16, body, 0)


def _zero_2d(ref, rows, width):
    z = jnp.zeros((16,), ref.dtype)

    def body(i, carry):
        for j in range(width // 16):
            ref[i, pl.ds(16 * j, 16)] = z
        return carry

    lax.fori_loop(0, rows, body, 0)


@functools.lru_cache(maxsize=None)
def _build_sc_deg():
    return pl.kernel(
        _sc_deg_body,
        out_type=jax.ShapeDtypeStruct((NW, NP), jnp.float32),
        mesh=_mesh(),
        compiler_params=pltpu.CompilerParams(needs_layout_passes=False),
        scratch_types=[
            pltpu.VMEM((EPW,), jnp.int32),
            pltpu.VMEM((NP,), jnp.float32),
        ],
    )


def _sc_deg(dst2):
    return _build_sc_deg()(dst2)


def _sc_deg_body(dst_hbm, out_hbm, idx_v, acc_v):
    wid = _wid()
    pltpu.sync_copy(dst_hbm.at[pl.ds(wid * EPW, EPW)], idx_v)
    _zero_1d(acc_v, NP)
    ones = jnp.ones((16,), jnp.float32)

    def step(i, carry):
        idx = idx_v[pl.ds(i * 16, 16)]
        plsc.addupdate_scatter(acc_v, [idx], ones)
        return carry

    lax.fori_loop(0, EPW // 16, step, 0)
    pltpu.sync_copy(acc_v, out_hbm.at[wid])


@functools.lru_cache(maxsize=None)
def _build_sc_agg1():
    return pl.kernel(
        _sc_agg1_body,
        out_type=jax.ShapeDtypeStruct((NW, NP), jnp.float32),
        mesh=_mesh(),
        compiler_params=pltpu.CompilerParams(needs_layout_passes=False),
        scratch_types=[
            pltpu.VMEM((EPW,), jnp.int32),
            pltpu.VMEM((EPW,), jnp.int32),
            pltpu.VMEM((NP,), jnp.float32),
            pltpu.VMEM((NP,), jnp.float32),
        ],
    )


def _sc_agg1(src2, dst2, u1):
    return _build_sc_agg1()(src2, dst2, u1)


def _sc_agg1_body(src_hbm, dst_hbm, u1_hbm, out_hbm, src_v, dst_v, u1_v, acc_v):
    wid = _wid()
    pltpu.sync_copy(src_hbm.at[pl.ds(wid * EPW, EPW)], src_v)
    pltpu.sync_copy(dst_hbm.at[pl.ds(wid * EPW, EPW)], dst_v)
    pltpu.sync_copy(u1_hbm, u1_v)
    _zero_1d(acc_v, NP)

    def step(i, carry):
        s = src_v[pl.ds(i * 16, 16)]
        d = dst_v[pl.ds(i * 16, 16)]
        vals = plsc.load_gather(u1_v, [s])
        plsc.addupdate_scatter(acc_v, [d], vals)
        return carry

    lax.fori_loop(0, EPW // 16, step, 0)
    pltpu.sync_copy(acc_v, out_hbm.at[wid])


@functools.lru_cache(maxsize=None)
def _build_sc_agg2():
    return pl.kernel(
        _sc_agg2_body,
        out_type=jax.ShapeDtypeStruct((NC, NP, HP), jnp.float32),
        mesh=_mesh(),
        compiler_params=pltpu.CompilerParams(needs_layout_passes=False),
        scratch_types=[
            pltpu.VMEM((CH,), jnp.int32),
            pltpu.VMEM((CH,), jnp.int32),
            pltpu.VMEM((CH, HP), jnp.float32),
            pltpu.VMEM((CH, HP), jnp.float32),
            pltpu.VMEM_SHARED((NP, HP), jnp.float32),
            pltpu.SemaphoreType.DMA,
        ],
    )


def _sc_agg2(src2, dst2, u2):
    return _build_sc_agg2()(src2, dst2, u2)


def _sc_agg2_body(src_hbm, dst_hbm, u2_hbm, out_hbm, sidx_v, didx_v, rows_v,
                  zbuf_v, acc_sp, sem):
    cid = lax.axis_index("c")
    sid = lax.axis_index("s")
    wid = sid * NC + cid
    rows_per_tile = NP // NS
    _zero_2d(zbuf_v, CH, HP)
    for j in range(rows_per_tile // CH):
        pltpu.sync_copy(zbuf_v, acc_sp.at[pl.ds(sid * rows_per_tile + j * CH, CH)])
    plsc.subcore_barrier()

    def step(i, carry):
        base = wid * EPW + i * CH
        pltpu.sync_copy(src_hbm.at[pl.ds(base, CH)], sidx_v)
        pltpu.sync_copy(dst_hbm.at[pl.ds(base, CH)], didx_v)
        pltpu.async_copy(u2_hbm.at[sidx_v], rows_v, sem).wait()
        pltpu.sync_copy(rows_v, acc_sp.at[didx_v], add=True)
        return carry

    lax.fori_loop(0, EPW // CH, step, 0)
    plsc.subcore_barrier()
    pltpu.sync_copy(
        acc_sp.at[pl.ds(sid * rows_per_tile, rows_per_tile)],
        out_hbm.at[cid, pl.ds(sid * rows_per_tile, rows_per_tile)],
    )


@functools.lru_cache(maxsize=None)
def _build_sc_gather():
    return pl.kernel(
        _sc_gather_body,
        out_type=(
            jax.ShapeDtypeStruct((E2, HP), jnp.float32),
            jax.ShapeDtypeStruct((E2, HP), jnp.float32),
        ),
        mesh=_mesh(),
        compiler_params=pltpu.CompilerParams(needs_layout_passes=False),
        scratch_types=[
            pltpu.VMEM((CH,), jnp.int32),
            pltpu.VMEM((CH,), jnp.int32),
            pltpu.VMEM((CH, HP), jnp.float32),
            pltpu.VMEM((CH, HP), jnp.float32),
            pltpu.SemaphoreType.DMA,
            pltpu.SemaphoreType.DMA,
        ],
    )


def _sc_gather(src2, dst2, h2):
    return _build_sc_gather()(src2, dst2, h2)


def _sc_gather_body(src_hbm, dst_hbm, h2_hbm, snd_hbm, rcv_hbm, sidx_v, didx_v,
                    srows_v, drows_v, sem1, sem2):
    wid = _wid()

    def step(i, carry):
        base = wid * EPW + i * CH
        pltpu.sync_copy(src_hbm.at[pl.ds(base, CH)], sidx_v)
        pltpu.sync_copy(dst_hbm.at[pl.ds(base, CH)], didx_v)
        c1 = pltpu.async_copy(h2_hbm.at[sidx_v], srows_v, sem1)
        c2 = pltpu.async_copy(h2_hbm.at[didx_v], drows_v, sem2)
        c1.wait()
        c2.wait()
        pltpu.sync_copy(srows_v, snd_hbm.at[pl.ds(base, CH)])
        pltpu.sync_copy(drows_v, rcv_hbm.at[pl.ds(base, CH)])
        return carry

    lax.fori_loop(0, EPW // CH, step, 0)


def _tc_a_body(degp_ref, x_ref, dinv_ref, u1_ref):
    deg = jnp.sum(degp_ref[...], axis=0, keepdims=True) + 1.0
    dinv = lax.rsqrt(deg)
    dinv_ref[...] = dinv
    u1_ref[...] = x_ref[...] * dinv


def _tc_b1_body(aggp_ref, u1_ref, dinv_ref, y_ref):
    agg = jnp.sum(aggp_ref[...], axis=0, keepdims=True)
    y_ref[...] = dinv_ref[...] * (agg + u1_ref[...])


def _tc_b2_body(y_ref, dinv_ref, w1_ref, b1_ref, g1_ref, be1_ref, w2_ref,
                u2_ref):
    g1s = g1_ref[...] * BN_S
    h1 = jnp.maximum((y_ref[...] * w1_ref[...] + b1_ref[...]) * g1s
                     + be1_ref[...], 0.0)
    u2 = dinv_ref[...] * jnp.dot(
        h1, w2_ref[...], preferred_element_type=jnp.float32)
    u2_ref[...] = jnp.concatenate(
        [u2, jnp.zeros((NP, HP - H), jnp.float32)], axis=1)


def _tc_c_body(p0_ref, p1_ref, u2_ref, dinv_ref, b2_ref, g2_ref, be2_ref,
               h2_ref):
    g2s = g2_ref[...] * BN_S
    agg = (p0_ref[...] + p1_ref[...] + u2_ref[...])[:, :H]
    h2 = jnp.maximum(
        (dinv_ref[...] * agg + b2_ref[...]) * g2s + be2_ref[...], 0.0)
    h2_ref[...] = jnp.concatenate(
        [h2, jnp.zeros((NP, HP - H), jnp.float32)], axis=1)


def _lstm_body(snd_ref, rcv_ref, ea_ref,
               ws_ref, wr_ref, we_ref, b_ref,
               wi_ref, wf_ref, wg_ref, wo_ref,
               hs_s,
               gi_s, gf_s, gg_s, go_s, h_s, c_s):
    g = (jnp.dot(snd_ref[...], ws_ref[...], preferred_element_type=jnp.float32)
         + jnp.dot(rcv_ref[...], wr_ref[...],
                   preferred_element_type=jnp.float32)
         + jnp.dot(ea_ref[...], we_ref[...],
                   preferred_element_type=jnp.float32)
         + b_ref[...])
    gi_s[...] = lax.slice(g, (0, 0), (T, LH))
    gf_s[...] = lax.slice(g, (0, LH), (T, 2 * LH))
    gg_s[...] = lax.slice(g, (0, 2 * LH), (T, 3 * LH))
    go_s[...] = lax.slice(g, (0, 3 * LH), (T, 4 * LH))

    @pl.when(pl.program_id(0) == 0)
    def _():
        h_s[...] = jnp.zeros((1, LH), jnp.float32)
        c_s[...] = jnp.zeros((1, LH), jnp.float32)

    wi = wi_ref[...]
    wf = wf_ref[...]
    wg = wg_ref[...]
    wo = wo_ref[...]

    def mv(hb, w):
        p = hb * w
        for sz in (32, 16, 8):
            p = (lax.slice(p, (0, 0), (sz, LH))
                 + lax.slice(p, (sz, 0), (2 * sz, LH)))
        return jnp.sum(p, axis=0, keepdims=True)

    def step(t, carry):
        h, c, hb = carry
        zi = gi_s[pl.ds(t, 1), :] + mv(hb, wi)
        zf = gf_s[pl.ds(t, 1), :] + mv(hb, wf)
        zg = gg_s[pl.ds(t, 1), :] + mv(hb, wg)
        zo = go_s[pl.ds(t, 1), :] + mv(hb, wo)
        ig = 0.5 * jnp.tanh(0.5 * zi) + 0.5
        fg = 0.5 * jnp.tanh(0.5 * zf) + 0.5
        gg = jnp.tanh(zg)
        og = 0.5 * jnp.tanh(0.5 * zo) + 0.5
        c = fg * c + ig * gg
        h = og * jnp.tanh(c)
        hs_s[pl.ds(t, 1), :] = h
        hb = jnp.broadcast_to(jnp.reshape(h, (LH, 1)), (LH, LH))
        return (h, c, hb)

    h, c, _ = lax.fori_loop(
        0, T, step,
        (h_s[...], c_s[...],
         jnp.broadcast_to(jnp.reshape(h_s[...], (LH, 1)), (LH, LH))))
    h_s[...] = h
    c_s[...] = c


def _mlp_body(hs_ref, wl1_ref, bl1_ref, wl2_ref, bl2_ref, out_ref):
    m = jnp.maximum(
        jnp.dot(hs_ref[...], wl1_ref[...], preferred_element_type=jnp.float32)
        + bl1_ref[...], 0.0)
    out_ref[...] = (jnp.dot(m, wl2_ref[...], preferred_element_type=jnp.float32)
                    + bl2_ref[...])


def _full_spec(shape):
    return pl.BlockSpec(shape, lambda i: tuple(0 for _ in shape))


def kernel(x, edge_index, edge_attr, W1, b1, g1, be1, W2, b2, g2, be2, Wih,
           Whh, bih, bhh, Wl1, bl1, Wl2, bl2):
    f32 = jnp.float32
    src = edge_index[0].astype(jnp.int32)
    dst = edge_index[1].astype(jnp.int32)
    padi = jnp.full((E2 - E,), N, jnp.int32)
    src2 = jnp.concatenate([src, padi])
    dst2 = jnp.concatenate([dst, padi])

    x_row = jnp.zeros((1, NP), f32).at[0, :N].set(x[:, 0])

    degp = _sc_deg(dst2)
    dinv_row, u1_row = pl.pallas_call(
        _tc_a_body,
        out_shape=(jax.ShapeDtypeStruct((1, NP), f32),
                   jax.ShapeDtypeStruct((1, NP), f32)),
    )(degp, x_row)

    aggp1 = _sc_agg1(src2, dst2, u1_row.reshape(NP))
    y_row = pl.pallas_call(
        _tc_b1_body,
        out_shape=jax.ShapeDtypeStruct((1, NP), f32),
    )(aggp1, u1_row, dinv_row)

    y_col = y_row.reshape(NP, 1)
    dinv_col = dinv_row.reshape(NP, 1)

    u2 = pl.pallas_call(
        _tc_b2_body,
        out_shape=jax.ShapeDtypeStruct((NP, HP), f32),
    )(y_col, dinv_col, W1.reshape(1, H), b1.reshape(1, H), g1.reshape(1, H),
      be1.reshape(1, H), W2)

    aggp2 = _sc_agg2(src2, dst2, u2)
    h2 = pl.pallas_call(
        _tc_c_body,
        out_shape=jax.ShapeDtypeStruct((NP, HP), f32),
    )(aggp2[0], aggp2[1], u2, dinv_col, b2.reshape(1, H), g2.reshape(1, H),
      be2.reshape(1, H))

    snd2, rcv2 = _sc_gather(src2, dst2, h2)
    snd = lax.slice(snd2, (0, 0), (E, HP))
    rcv = lax.slice(rcv2, (0, 0), (E, HP))

    WihT = Wih.T
    ws = jnp.concatenate(
        [lax.slice(WihT, (0, 0), (H, 4 * LH)),
         jnp.zeros((HP - H, 4 * LH), jnp.float32)])
    wr = jnp.concatenate(
        [lax.slice(WihT, (H, 0), (2 * H, 4 * LH)),
         jnp.zeros((HP - H, 4 * LH), jnp.float32)])
    we = lax.slice(WihT, (2 * H, 0), (2 * H + 2, 4 * LH))
    b = (bih + bhh).reshape(1, 4 * LH)
    WhhT = Whh.T
    wi = lax.slice(WhhT, (0, 0), (LH, LH))
    wf = lax.slice(WhhT, (0, LH), (LH, 2 * LH))
    wg = lax.slice(WhhT, (0, 2 * LH), (LH, 3 * LH))
    wo = lax.slice(WhhT, (0, 3 * LH), (LH, 4 * LH))

    grid = E // T
    edge_spec = pl.BlockSpec((T, HP), lambda i: (i, 0))
    hs = pl.pallas_call(
        _lstm_body,
        grid=(grid,),
        in_specs=[
            edge_spec, edge_spec, pl.BlockSpec((T, 2), lambda i: (i, 0)),
            _full_spec((HP, 4 * LH)), _full_spec((HP, 4 * LH)),
            _full_spec((2, 4 * LH)), _full_spec((1, 4 * LH)),
            _full_spec((LH, LH)), _full_spec((LH, LH)),
            _full_spec((LH, LH)), _full_spec((LH, LH)),
        ],
        out_specs=pl.BlockSpec((T, LH), lambda i: (i, 0)),
        out_shape=jax.ShapeDtypeStruct((E, LH), f32),
        scratch_shapes=[
            pltpu.VMEM((T, LH), f32), pltpu.VMEM((T, LH), f32),
            pltpu.VMEM((T, LH), f32), pltpu.VMEM((T, LH), f32),
            pltpu.VMEM((1, LH), f32), pltpu.VMEM((1, LH), f32),
        ],
        compiler_params=pltpu.CompilerParams(
            dimension_semantics=("arbitrary",)),
    )(snd, rcv, edge_attr, ws, wr, we, b, wi, wf, wg, wo)

    TM = 8000
    out = pl.pallas_call(
        _mlp_body,
        grid=(E // TM,),
        in_specs=[
            pl.BlockSpec((TM, LH), lambda i: (i, 0)),
            _full_spec((LH, LH // 2)), _full_spec((1, LH // 2)),
            _full_spec((LH // 2, 1)), _full_spec((1, 1)),
        ],
        out_specs=pl.BlockSpec((TM, 1), lambda i: (i, 0)),
        out_shape=jax.ShapeDtypeStruct((E, 1), f32),
        compiler_params=pltpu.CompilerParams(
            dimension_semantics=("arbitrary",)),
    )(hs, Wl1.T, bl1.reshape(1, LH // 2), Wl2.T, bl2.reshape(1, 1))

    return out.reshape(-1)
